# Initial kernel scaffold; baseline (speedup 1.0000x reference)
import jax
import jax.numpy as jnp
from jax.experimental import pallas as pl


def kernel(edge_type_matrix, edge_embedding_weight):
    raise NotImplementedError("write your pallas kernel here")


# speedup vs baseline: 17.8343x; 17.8343x over previous
import functools

import jax
import jax.numpy as jnp
from jax import lax
from jax.experimental import pallas as pl
from jax.experimental.pallas import tpu as pltpu
from jax.experimental.pallas import tpu_sc as plsc

NUM_HEADS = 16
NUM_TYPES = 32
S = 2048
N = S * S

NC = 2
NS = 16
L = 16
NW = NC * NS
PER_W = N // NW
CHUNK = 2048
N_CHUNKS = PER_W // CHUNK
GROUPS = CHUNK // L

_mesh = plsc.VectorSubcoreMesh(core_axis_name="c", subcore_axis_name="s")


@functools.partial(
    pl.kernel,
    out_type=jax.ShapeDtypeStruct((NUM_HEADS, N), jnp.float32),
    mesh=_mesh,
    scratch_types=[
        pltpu.VMEM((NUM_HEADS * NUM_TYPES,), jnp.float32),
        pltpu.VMEM((CHUNK,), jnp.int32),
        pltpu.VMEM((NUM_HEADS, CHUNK), jnp.float32),
    ],
    compiler_params=pltpu.CompilerParams(needs_layout_passes=False),
)
def _edge_bias_sc(idx_hbm, tbl_hbm, out_hbm, tbl_v, idx_v, out_v):
    wid = lax.axis_index("s") * NC + lax.axis_index("c")
    pltpu.sync_copy(tbl_hbm, tbl_v)

    def chunk_body(c, carry):
        base = wid * PER_W + c * CHUNK
        pltpu.sync_copy(idx_hbm.at[pl.ds(base, CHUNK)], idx_v)

        def grp_body(g, carry2):
            idx = idx_v[pl.ds(g * L, L)]
            for h in range(NUM_HEADS):
                vals = plsc.load_gather(tbl_v, [idx + h * NUM_TYPES])
                out_v[h, pl.ds(g * L, L)] = vals
            return carry2

        lax.fori_loop(0, GROUPS, grp_body, 0, unroll=2)

        for h in range(NUM_HEADS):
            pltpu.sync_copy(out_v.at[h], out_hbm.at[h, pl.ds(base, CHUNK)])
        return carry

    lax.fori_loop(0, N_CHUNKS, chunk_body, 0)


def kernel(edge_type_matrix, edge_embedding_weight):
    idx = edge_type_matrix.reshape(-1).astype(jnp.int32)
    tbl = edge_embedding_weight.T.reshape(-1)
    out = _edge_bias_sc(idx, tbl)
    return out.reshape(NUM_HEADS, S, S)


# speedup vs baseline: 48.2978x; 2.7081x over previous
import functools

import jax
import jax.numpy as jnp
from jax import lax
from jax.experimental import pallas as pl
from jax.experimental.pallas import tpu as pltpu
from jax.experimental.pallas import tpu_sc as plsc

NUM_HEADS = 16
NUM_TYPES = 32
S = 2048
N = S * S

NC = 2
NS = 16
L = 16
NW = NC * NS
PER_W = N // NW
CHUNK = 2048
N_CHUNKS = PER_W // CHUNK
GROUPS = CHUNK // L
NBUF = 2

_mesh = plsc.VectorSubcoreMesh(core_axis_name="c", subcore_axis_name="s")


@functools.partial(
    pl.kernel,
    out_type=jax.ShapeDtypeStruct((NUM_HEADS, N), jnp.float32),
    mesh=_mesh,
    scratch_types=[
        pltpu.VMEM((NUM_HEADS * NUM_TYPES,), jnp.float32),
        pltpu.VMEM((NBUF, CHUNK), jnp.int32),
        pltpu.VMEM((NBUF, NUM_HEADS, CHUNK), jnp.float32),
        pltpu.SemaphoreType.DMA,
        pltpu.SemaphoreType.DMA,
    ],
    compiler_params=pltpu.CompilerParams(needs_layout_passes=False),
)
def _edge_bias_sc(idx_hbm, tbl_hbm, out_hbm, tbl_v, idx_v, out_v, in_sem,
                  out_sem):
    wid = lax.axis_index("s") * NC + lax.axis_index("c")
    base0 = wid * PER_W
    pltpu.sync_copy(tbl_hbm, tbl_v)
    pltpu.async_copy(idx_hbm.at[pl.ds(base0, CHUNK)], idx_v.at[0], in_sem)

    def pair_body(p, carry):
        for b in range(NBUF):
            c = p * NBUF + b
            base = base0 + c * CHUNK
            nb = (b + 1) % NBUF

            @pl.when(c + 1 < N_CHUNKS)
            def _prefetch():
                pltpu.async_copy(idx_hbm.at[pl.ds(base + CHUNK, CHUNK)],
                                 idx_v.at[nb], in_sem)

            pltpu.make_async_copy(idx_hbm.at[pl.ds(base, CHUNK)],
                                  idx_v.at[b], in_sem).wait()

            @pl.when(c >= NBUF)
            def _drain():
                pltpu.make_async_copy(out_v.at[b],
                                      out_hbm.at[:, pl.ds(base, CHUNK)],
                                      out_sem).wait()

            @plsc.parallel_loop(0, GROUPS, unroll=4)
            def grp_body(g):
                idx = idx_v[b, pl.ds(g * L, L)]
                for h in range(NUM_HEADS):
                    vals = plsc.load_gather(tbl_v, [idx + h * NUM_TYPES])
                    out_v[b, h, pl.ds(g * L, L)] = vals

            pltpu.async_copy(out_v.at[b], out_hbm.at[:, pl.ds(base, CHUNK)],
                             out_sem)
        return carry

    lax.fori_loop(0, N_CHUNKS // NBUF, pair_body, 0)
    for b in range(NBUF):
        pltpu.make_async_copy(out_v.at[b],
                              out_hbm.at[:, pl.ds(base0, CHUNK)],
                              out_sem).wait()


def kernel(edge_type_matrix, edge_embedding_weight):
    idx = edge_type_matrix.reshape(-1).astype(jnp.int32)
    tbl = edge_embedding_weight.T.reshape(-1)
    out = _edge_bias_sc(idx, tbl)
    return out.reshape(NUM_HEADS, S, S)


# speedup vs baseline: 124.3695x; 2.5751x over previous
import functools

import jax
import jax.numpy as jnp
from jax import lax
from jax.experimental import pallas as pl
from jax.experimental.pallas import tpu as pltpu
from jax.experimental.pallas import tpu_sc as plsc

NUM_HEADS = 16
NUM_TYPES = 32
S = 2048
N = S * S

NC = 2
NS = 16
L = 16
NW = NC * NS
TROWS = S // 8
TROWS_W = TROWS // NW
CW = 256
CHUNK = 8 * CW
N_CHUNKS = TROWS_W * (S // CW)
GROUPS = CHUNK // L
NBUF = 2

_mesh = plsc.VectorSubcoreMesh(core_axis_name="c", subcore_axis_name="s")


@functools.partial(
    pl.kernel,
    out_type=jax.ShapeDtypeStruct((NUM_HEADS, S, S), jnp.float32),
    mesh=_mesh,
    scratch_types=[
        pltpu.VMEM((NUM_HEADS * NUM_TYPES,), jnp.float32),
        pltpu.VMEM((NBUF, 8, CW), jnp.int32),
        pltpu.VMEM((NBUF, NUM_HEADS, 8, CW), jnp.float32),
        pltpu.SemaphoreType.DMA,
        pltpu.SemaphoreType.DMA,
    ],
    compiler_params=pltpu.CompilerParams(
        needs_layout_passes=False, use_tc_tiling_on_sc=True),
)
def _edge_bias_sc(idx_hbm, tbl_hbm, out_hbm, tbl_v, idx_v, out_v, in_sem,
                  out_sem):
    wid = lax.axis_index("s") * NC + lax.axis_index("c")
    row0 = wid * TROWS_W * 8
    cpr = S // CW

    def chunk_slices(c):
        r = row0 + (c // cpr) * 8
        col = (c % cpr) * CW
        return pl.ds(r, 8), pl.ds(col, CW)

    pltpu.sync_copy(tbl_hbm, tbl_v)
    r0, c0 = chunk_slices(0)
    pltpu.async_copy(idx_hbm.at[r0, c0], idx_v.at[0], in_sem)

    def pair_body(p, carry):
        for b in range(NBUF):
            c = p * NBUF + b
            rs, cs = chunk_slices(c)
            nb = (b + 1) % NBUF

            @pl.when(c + 1 < N_CHUNKS)
            def _prefetch():
                nrs, ncs = chunk_slices(c + 1)
                pltpu.async_copy(idx_hbm.at[nrs, ncs], idx_v.at[nb], in_sem)

            pltpu.make_async_copy(idx_hbm.at[rs, cs], idx_v.at[b],
                                  in_sem).wait()

            @pl.when(c >= NBUF)
            def _drain():
                pltpu.make_async_copy(out_v.at[b], out_hbm.at[:, rs, cs],
                                      out_sem).wait()

            @plsc.parallel_loop(0, GROUPS, unroll=4)
            def grp_body(g):
                row = g // (CW // L)
                col = (g % (CW // L)) * L
                idx = idx_v[b, row, pl.ds(col, L)]
                for h in range(NUM_HEADS):
                    vals = plsc.load_gather(tbl_v, [idx + h * NUM_TYPES])
                    out_v[b, h, row, pl.ds(col, L)] = vals

            pltpu.async_copy(out_v.at[b], out_hbm.at[:, rs, cs], out_sem)
        return carry

    lax.fori_loop(0, N_CHUNKS // NBUF, pair_body, 0)
    r0, c0 = chunk_slices(0)
    for b in range(NBUF):
        pltpu.make_async_copy(out_v.at[b], out_hbm.at[:, r0, c0],
                              out_sem).wait()


def kernel(edge_type_matrix, edge_embedding_weight):
    idx = edge_type_matrix.astype(jnp.int32)
    tbl = edge_embedding_weight.T.reshape(-1)
    return _edge_bias_sc(idx, tbl)
